# baseline (speedup 1.0000x reference)
import numpy as np

import jax
import jax.numpy as jnp
from jax.experimental import pallas as pl
from jax.experimental.pallas import tpu as pltpu


_t = np.arange(9)
_g = np.arange(4)
_dy, _dx = _t // 3, _t % 3
_dr, _dc = _g // 2, _g % 2

_x14 = np.arange(14)
_kval = ((_dr[None, :, None] + _dy[:, None, None]) * 32
         + 2 * _x14[None, None, :] + _dc[None, :, None] + _dx[:, None, None])
_A1 = np.ascontiguousarray(np.moveaxis(
    (np.arange(128)[None, None, :, None]
     == _kval[:, :, None, :]).astype(np.float32), 2, 1))

_x7 = np.arange(7)
_g2 = np.arange(2)
_pval = (2 * _x7[None, None, :] + _g2[None, :, None]
         + _dx[:, None, None])
_B2 = np.ascontiguousarray(
    ((np.arange(3)[None, :, None, None, None] == _dy[:, None, None, None, None])
     & (np.arange(16)[None, None, :, None, None]
        == _pval[:, None, None, :, :]))
    .astype(np.float32))


def _build_w1(w1):
    w = jnp.einsum('tkgx,tc->kgxc', _A1, w1)
    w = jnp.pad(w.reshape(128, 4, 224), ((0, 0), (0, 0), (16, 16)))
    return w.reshape(128, 1024).astype(jnp.bfloat16)


def _build_w2(w2):
    w = jnp.einsum('trpgx,tio->rpigxo', _B2, w2.reshape(9, 16, 32))
    w = jnp.pad(w.reshape(768, 2, 224), ((0, 0), (0, 0), (0, 32)))
    return w.reshape(768, 512).astype(jnp.bfloat16)


def _cnn_block_kernel(x_ref,
                      w1_ref, v1_ref,
                      w2_ref, v2_ref,
                      dw1_ref, db1_ref, scd_ref, shd_ref,
                      dw2_ref, db2_ref,
                      o_ref,
                      s2_ref,
                      f_ref):
    f32 = jnp.float32
    bf16 = jnp.bfloat16
    B = x_ref.shape[0]

    X = x_ref[...]
    zeros = jnp.zeros((B, 256), bf16)
    s2_ref[:, 0:256] = zeros
    s2_ref[:, 3840:4096] = zeros

    w1 = w1_ref[...]
    b1 = v1_ref[0:1]
    sc1 = v1_ref[1:2]
    sh1 = v1_ref[2:3]
    for i in range(14):
        P = X[:, 64 * i:64 * i + 128]
        M = jnp.dot(P, w1, preferred_element_type=f32)
        V = jnp.maximum(jnp.maximum(M[:, 0:256], M[:, 256:512]),
                        jnp.maximum(M[:, 512:768], M[:, 768:1024]))
        V = sc1 * jnp.maximum(V + b1, 0.0) + sh1
        s2_ref[:, 256 * (i + 1):256 * (i + 2)] = V.astype(bf16)

    w2 = w2_ref[...]
    b2 = v2_ref[0:1]
    sc2 = v2_ref[1:2]
    sh2 = v2_ref[2:3]
    for j in range(7):
        PA = s2_ref[:, 512 * j:512 * j + 768]
        PB = s2_ref[:, 512 * j + 256:512 * j + 1024]
        MA = jnp.dot(PA, w2, preferred_element_type=f32)
        MB = jnp.dot(PB, w2, preferred_element_type=f32)
        V2 = jnp.maximum(jnp.maximum(MA[:, 0:256], MA[:, 256:512]),
                         jnp.maximum(MB[:, 0:256], MB[:, 256:512]))
        V2 = sc2 * jnp.maximum(V2 + b2, 0.0) + sh2
        f_ref[:, 256 * j:256 * j + 256] = jnp.maximum(V2, 0.0)

    F = f_ref[...]
    H = jnp.dot(F, dw1_ref[...], preferred_element_type=f32) + db1_ref[...]
    H = jnp.maximum(scd_ref[...] * H + shd_ref[...], 0.0)
    Y = jnp.dot(H, dw2_ref[...], preferred_element_type=f32) + db2_ref[...]
    o_ref[...] = jnp.maximum(Y, 0.0)


def _forward(xp, *weight_args):
    N = xp.shape[0]
    B = 1024 if N % 1024 == 0 else N

    def full_spec(a):
        nd = a.ndim
        return pl.BlockSpec(a.shape, lambda n, _nd=nd: (0,) * _nd)

    return pl.pallas_call(
        _cnn_block_kernel,
        out_shape=jax.ShapeDtypeStruct((N, 10), jnp.float32),
        grid=(N // B,),
        in_specs=[pl.BlockSpec((B, 960), lambda n: (n, 0))]
                 + [full_spec(a) for a in weight_args],
        out_specs=pl.BlockSpec((B, 10), lambda n: (n, 0)),
        scratch_shapes=[pltpu.VMEM((B, 4096), jnp.bfloat16),
                        pltpu.VMEM((B, 1792), jnp.float32)],
        compiler_params=pltpu.CompilerParams(dimension_semantics=("parallel",)),
    )(xp, *weight_args)


def kernel(x, w1, b1, sc1, sh1, w2, b2, sc2, sh2, dw1, db1, scd, shd, dw2, db2):
    N = x.shape[0]

    xp = jnp.pad(x.astype(jnp.float32).reshape(N, 28, 28),
                 ((0, 0), (1, 1), (1, 3))).reshape(N, 960).astype(jnp.bfloat16)

    w1b = _build_w1(w1)
    w2b = _build_w2(w2)
    v1 = jnp.pad(jnp.tile(jnp.concatenate([b1, sc1, sh1], axis=0), (1, 14)),
                 ((0, 0), (16, 16)))
    v2 = jnp.pad(jnp.tile(jnp.concatenate([b2, sc2, sh2], axis=0), (1, 7)),
                 ((0, 0), (0, 32)))
    dw1p = jnp.pad(dw1.reshape(7, 224, 50),
                   ((0, 0), (0, 32), (0, 0))).reshape(1792, 50)

    weight_args = (w1b, v1, w2b, v2, dw1p, db1, scd, shd, dw2, db2)

    devs = jax.devices()
    if len(devs) >= 2 and N % 2048 == 0:
        mesh = jax.sharding.Mesh(np.array(devs[:2]), ('d',))
        P = jax.sharding.PartitionSpec
        fwd = jax.shard_map(_forward, mesh=mesh,
                            in_specs=(P('d', None),) + (P(None, None),) * 10,
                            out_specs=P('d', None), check_vma=False)
        return fwd(xp, *weight_args)
    return _forward(xp, *weight_args)


# speedup vs baseline: 2.7048x; 2.7048x over previous
import numpy as np

import jax
import jax.numpy as jnp
from jax.experimental import pallas as pl
from jax.experimental.pallas import tpu as pltpu


_t = np.arange(9)
_g = np.arange(4)
_dy, _dx = _t // 3, _t % 3
_dr, _dc = _g // 2, _g % 2

_x14 = np.arange(14)
_kval = ((_dr[None, :, None] + _dy[:, None, None]) * 32
         + 2 * _x14[None, None, :] + _dc[None, :, None] + _dx[:, None, None])
_A1 = np.ascontiguousarray(np.moveaxis(
    (np.arange(128)[None, None, :, None]
     == _kval[:, :, None, :]).astype(np.float32), 2, 1))

_x7 = np.arange(7)
_g2 = np.arange(2)
_pval = (2 * _x7[None, None, :] + _g2[None, :, None]
         + _dx[:, None, None])
_B2 = np.ascontiguousarray(
    ((np.arange(3)[None, :, None, None, None] == _dy[:, None, None, None, None])
     & (np.arange(16)[None, None, :, None, None]
        == _pval[:, None, None, :, :]))
    .astype(np.float32))


def _build_w1(w1):
    w = jnp.einsum('tkgx,tc->kgxc', _A1, w1)
    w = jnp.pad(w.reshape(128, 4, 224), ((0, 0), (0, 0), (16, 16)))
    return w.reshape(128, 1024).astype(jnp.bfloat16)


def _build_w2(w2):
    w = jnp.einsum('trpgx,tio->rpigxo', _B2, w2.reshape(9, 16, 32))
    w = jnp.pad(w.reshape(768, 2, 224), ((0, 0), (0, 0), (0, 32)))
    return w.reshape(768, 512).astype(jnp.bfloat16)


def _cnn_block_kernel(x_ref,
                      w1_ref, v1_ref,
                      w2_ref, v2_ref,
                      dw1_ref, db1_ref, scd_ref, shd_ref,
                      dw2_ref, db2_ref,
                      o_ref,
                      s2_ref,
                      f_ref):
    f32 = jnp.float32
    bf16 = jnp.bfloat16
    B = x_ref.shape[0]

    X = x_ref[...]
    zeros = jnp.zeros((B, 256), bf16)
    s2_ref[:, 0:256] = zeros
    s2_ref[:, 3840:4096] = zeros

    w1 = w1_ref[...]
    b1 = v1_ref[0:1]
    sc1 = v1_ref[1:2]
    sh1 = v1_ref[2:3]
    w2 = w2_ref[...]
    b2 = v2_ref[0:1]
    sc2 = v2_ref[1:2]
    sh2 = v2_ref[2:3]

    def conv1_pair(i):
        P = X[:, 64 * i:64 * i + 128]
        M = jnp.dot(P, w1, preferred_element_type=f32)
        V = jnp.maximum(jnp.maximum(M[:, 0:256], M[:, 256:512]),
                        jnp.maximum(M[:, 512:768], M[:, 768:1024]))
        V = sc1 * jnp.maximum(V + b1, 0.0) + sh1
        s2_ref[:, 256 * (i + 1):256 * (i + 2)] = V.astype(bf16)

    def conv2_pair(j):
        PA = s2_ref[:, 512 * j:512 * j + 768]
        PB = s2_ref[:, 512 * j + 256:512 * j + 1024]
        MA = jnp.dot(PA, w2, preferred_element_type=f32)
        MB = jnp.dot(PB, w2, preferred_element_type=f32)
        V2 = jnp.maximum(jnp.maximum(MA[:, 0:256], MA[:, 256:512]),
                         jnp.maximum(MB[:, 0:256], MB[:, 256:512]))
        V2 = sc2 * jnp.maximum(V2 + b2, 0.0) + sh2
        f_ref[:, 256 * j:256 * j + 256] = jnp.maximum(V2, 0.0)

    conv1_pair(0)
    conv1_pair(1)
    conv1_pair(2)
    for j in range(5):
        conv2_pair(j)
        conv1_pair(2 * j + 3)
        conv1_pair(2 * j + 4)
    conv2_pair(5)
    conv1_pair(13)
    conv2_pair(6)

    F = f_ref[...]
    H = jnp.dot(F, dw1_ref[...], preferred_element_type=f32) + db1_ref[...]
    H = jnp.maximum(scd_ref[...] * H + shd_ref[...], 0.0)
    Y = jnp.dot(H, dw2_ref[...], preferred_element_type=f32) + db2_ref[...]
    o_ref[...] = jnp.maximum(Y, 0.0)


def _forward(xp, *weight_args):
    N = xp.shape[0]
    B = 1024 if N % 1024 == 0 else N

    def full_spec(a):
        nd = a.ndim
        return pl.BlockSpec(a.shape, lambda n, _nd=nd: (0,) * _nd)

    return pl.pallas_call(
        _cnn_block_kernel,
        out_shape=jax.ShapeDtypeStruct((N, 10), jnp.float32),
        grid=(N // B,),
        in_specs=[pl.BlockSpec((B, 960), lambda n: (n, 0))]
                 + [full_spec(a) for a in weight_args],
        out_specs=pl.BlockSpec((B, 10), lambda n: (n, 0)),
        scratch_shapes=[pltpu.VMEM((B, 4096), jnp.bfloat16),
                        pltpu.VMEM((B, 1792), jnp.float32)],
        compiler_params=pltpu.CompilerParams(dimension_semantics=("parallel",)),
    )(xp, *weight_args)


def kernel(x, w1, b1, sc1, sh1, w2, b2, sc2, sh2, dw1, db1, scd, shd, dw2, db2):
    N = x.shape[0]

    xp = jnp.pad(x.astype(jnp.float32).reshape(N, 28, 28),
                 ((0, 0), (1, 1), (1, 3))).reshape(N, 960).astype(jnp.bfloat16)

    w1b = _build_w1(w1)
    w2b = _build_w2(w2)
    v1 = jnp.pad(jnp.tile(jnp.concatenate([b1, sc1, sh1], axis=0), (1, 14)),
                 ((0, 0), (16, 16)))
    v2 = jnp.pad(jnp.tile(jnp.concatenate([b2, sc2, sh2], axis=0), (1, 7)),
                 ((0, 0), (0, 32)))
    dw1p = jnp.pad(dw1.reshape(7, 224, 50),
                   ((0, 0), (0, 32), (0, 0))).reshape(1792, 50)

    weight_args = (w1b, v1, w2b, v2, dw1p, db1, scd, shd, dw2, db2)

    return _forward(xp, *weight_args)


# speedup vs baseline: 2.8044x; 1.0368x over previous
import numpy as np

import jax
import jax.numpy as jnp
from jax.experimental import pallas as pl
from jax.experimental.pallas import tpu as pltpu


_t = np.arange(9)
_g = np.arange(4)
_dy, _dx = _t // 3, _t % 3
_dr, _dc = _g // 2, _g % 2

_x14 = np.arange(14)
_kval = ((_dr[None, :, None] + _dy[:, None, None]) * 32
         + 2 * _x14[None, None, :] + _dc[None, :, None] + _dx[:, None, None])
_A1 = np.ascontiguousarray(np.moveaxis(
    (np.arange(128)[None, None, :, None]
     == _kval[:, :, None, :]).astype(np.float32), 2, 1))

_x7 = np.arange(7)
_g2 = np.arange(2)
_pval = (2 * _x7[None, None, :] + _g2[None, :, None]
         + _dx[:, None, None])
_B2 = np.ascontiguousarray(
    ((np.arange(3)[None, :, None, None, None] == _dy[:, None, None, None, None])
     & (np.arange(16)[None, None, :, None, None]
        == _pval[:, None, None, :, :]))
    .astype(np.float32))


def _build_w1(w1):
    w = jnp.einsum('tkgx,tc->kgxc', _A1, w1)
    w = jnp.pad(w.reshape(128, 4, 224), ((0, 0), (0, 0), (16, 16)))
    return w.reshape(128, 1024).astype(jnp.bfloat16)


def _build_w2(w2):
    w = jnp.einsum('trpgx,tio->rpigxo', _B2, w2.reshape(9, 16, 32))
    w = jnp.pad(w.reshape(768, 2, 224), ((0, 0), (0, 0), (0, 32)))
    return w.reshape(768, 512).astype(jnp.bfloat16)


def _cnn_block_kernel(x_ref,
                      w1_ref, v1_ref,
                      w2_ref, v2_ref,
                      dw1_ref, db1_ref, scd_ref, shd_ref,
                      dw2_ref, db2_ref,
                      o_ref,
                      s2_ref,
                      f_ref):
    f32 = jnp.float32
    bf16 = jnp.bfloat16
    B = x_ref.shape[0]

    X = x_ref[...]
    zeros = jnp.zeros((B, 256), bf16)
    s2_ref[:, 0:256] = zeros
    s2_ref[:, 3840:4096] = zeros

    w1 = w1_ref[...]
    b1 = v1_ref[0:1]
    sc1 = v1_ref[1:2]
    sh1 = v1_ref[2:3]
    w2 = w2_ref[...]
    b2 = v2_ref[0:1]
    sc2 = v2_ref[1:2]
    sh2 = v2_ref[2:3]

    def conv1_pair(i):
        P = X[:, 64 * i:64 * i + 128]
        M = jnp.dot(P, w1, preferred_element_type=f32)
        V = jnp.maximum(jnp.maximum(M[:, 0:256], M[:, 256:512]),
                        jnp.maximum(M[:, 512:768], M[:, 768:1024]))
        V = sc1 * jnp.maximum(V + b1, 0.0) + sh1
        s2_ref[:, 256 * (i + 1):256 * (i + 2)] = V.astype(bf16)

    def conv2_pair(j):
        PA = s2_ref[:, 512 * j:512 * j + 768]
        PB = s2_ref[:, 512 * j + 256:512 * j + 1024]
        MA = jnp.dot(PA, w2, preferred_element_type=f32)
        MB = jnp.dot(PB, w2, preferred_element_type=f32)
        V2 = jnp.maximum(jnp.maximum(MA[:, 0:256], MA[:, 256:512]),
                         jnp.maximum(MB[:, 0:256], MB[:, 256:512]))
        V2 = sc2 * jnp.maximum(V2 + b2, 0.0) + sh2
        f_ref[:, 256 * j:256 * j + 256] = jnp.maximum(V2, 0.0)

    for i in range(14):
        conv1_pair(i)
    for j in range(7):
        conv2_pair(j)

    F = f_ref[...]
    H = jnp.dot(F, dw1_ref[...], preferred_element_type=f32) + db1_ref[...]
    H = jnp.maximum(scd_ref[...] * H + shd_ref[...], 0.0)
    Y = jnp.dot(H, dw2_ref[...], preferred_element_type=f32) + db2_ref[...]
    o_ref[...] = jnp.maximum(Y, 0.0)


def _forward(xp, *weight_args):
    N = xp.shape[0]
    B = 1024 if N % 1024 == 0 else N

    def full_spec(a):
        nd = a.ndim
        return pl.BlockSpec(a.shape, lambda n, _nd=nd: (0,) * _nd)

    return pl.pallas_call(
        _cnn_block_kernel,
        out_shape=jax.ShapeDtypeStruct((N, 10), jnp.float32),
        grid=(N // B,),
        in_specs=[pl.BlockSpec((B, 960), lambda n: (n, 0))]
                 + [full_spec(a) for a in weight_args],
        out_specs=pl.BlockSpec((B, 10), lambda n: (n, 0)),
        scratch_shapes=[pltpu.VMEM((B, 4096), jnp.bfloat16),
                        pltpu.VMEM((B, 1792), jnp.float32)],
        compiler_params=pltpu.CompilerParams(dimension_semantics=("parallel",)),
    )(xp, *weight_args)


def kernel(x, w1, b1, sc1, sh1, w2, b2, sc2, sh2, dw1, db1, scd, shd, dw2, db2):
    N = x.shape[0]

    xp = jnp.pad(x.astype(jnp.float32).reshape(N, 28, 28),
                 ((0, 0), (1, 1), (1, 3))).reshape(N, 960).astype(jnp.bfloat16)

    w1b = _build_w1(w1)
    w2b = _build_w2(w2)
    v1 = jnp.pad(jnp.tile(jnp.concatenate([b1, sc1, sh1], axis=0), (1, 14)),
                 ((0, 0), (16, 16)))
    v2 = jnp.pad(jnp.tile(jnp.concatenate([b2, sc2, sh2], axis=0), (1, 7)),
                 ((0, 0), (0, 32)))
    dw1p = jnp.pad(dw1.reshape(7, 224, 50),
                   ((0, 0), (0, 32), (0, 0))).reshape(1792, 50)

    weight_args = (w1b, v1, w2b, v2, dw1p, db1, scd, shd, dw2, db2)

    return _forward(xp, *weight_args)
